# baseline (speedup 1.0000x reference)
import functools

import jax
import jax.numpy as jnp
from jax import lax
from jax.experimental import pallas as pl
from jax.experimental.pallas import tpu as pltpu
from jax.experimental.pallas import tpu_sc as plsc

NC = 2
NS = 16
NW = NC * NS
CH = 128


def _sc_neighbors(B, K, user_idxs, item_idxs, uidx_t, iidx_t):
  bpw = B // NW
  mesh = plsc.VectorSubcoreMesh(core_axis_name="c", subcore_axis_name="s")

  @functools.partial(
      pl.kernel,
      mesh=mesh,
      compiler_params=pltpu.CompilerParams(use_tc_tiling_on_sc=False),
      out_type=[
          jax.ShapeDtypeStruct((B, K), jnp.int32),
          jax.ShapeDtypeStruct((B, K), jnp.int32),
      ],
      scratch_types=[
          pltpu.VMEM((bpw,), jnp.int32),
          pltpu.VMEM((bpw,), jnp.int32),
          pltpu.VMEM((bpw, K), jnp.int32),
          pltpu.VMEM((bpw, K), jnp.int32),
          pltpu.SemaphoreType.DMA,
      ],
  )
  def k(uids_h, iids_h, uidx_h, iidx_h, un_out, in_out,
        ub_v, ib_v, un_v, in_v, sem):
    wid = lax.axis_index("s") * NC + lax.axis_index("c")
    base = wid * bpw
    pltpu.sync_copy(uids_h.at[pl.ds(base, bpw)], ub_v)
    pltpu.sync_copy(iids_h.at[pl.ds(base, bpw)], ib_v)
    c1 = pltpu.async_copy(uidx_h.at[ub_v], un_v, sem)
    c2 = pltpu.async_copy(iidx_h.at[ib_v], in_v, sem)
    c1.wait()
    c2.wait()
    pltpu.sync_copy(un_v, un_out.at[pl.ds(base, bpw)])
    pltpu.sync_copy(in_v, in_out.at[pl.ds(base, bpw)])

  return k(user_idxs, item_idxs, uidx_t, iidx_t)


def _sc_rows(B, K, D, un_flat, in_flat, uemb_t, uscr_t, iemb_t, iscr_t):
  R = B * K
  rpw = R // NW
  n_ch = rpw // CH
  mesh = plsc.VectorSubcoreMesh(core_axis_name="c", subcore_axis_name="s")

  @functools.partial(
      pl.kernel,
      mesh=mesh,
      compiler_params=pltpu.CompilerParams(use_tc_tiling_on_sc=False),
      out_type=[
          jax.ShapeDtypeStruct((R, D), jnp.float32),
      ],
      scratch_types=[
          pltpu.VMEM((CH,), jnp.int32),
          pltpu.VMEM((CH,), jnp.int32),
          pltpu.VMEM((CH, D), jnp.float32),
          pltpu.VMEM((CH, K), jnp.float32),
          pltpu.VMEM((CH, D), jnp.float32),
          pltpu.VMEM((CH, K), jnp.float32),
          pltpu.SemaphoreType.DMA,
      ],
  )
  def k(un_h, uemb_h, ue_out, ui_v, ii_v, ue_b, us_b, ie_b, is_b, sem):
    wid = lax.axis_index("s") * NC + lax.axis_index("c")
    base = wid * rpw

    for p in range(n_ch):
      off = base + p * CH
      pltpu.sync_copy(un_h.at[pl.ds(off, CH)], ui_v)
      pltpu.sync_copy(uemb_h.at[ui_v], ue_b)
      pltpu.sync_copy(ue_b, ue_out.at[pl.ds(off, CH)])

  return k(un_flat, uemb_t)


def _tc_body(us_r, ue_r, is_r, ie_r, w1_r, b1_r, w2_r, b2_r, o_r, acc_r):
  bb = us_r.shape[0]
  wv = jnp.dot(w1_r[...], w2_r[...], preferred_element_type=jnp.float32)
  c = jnp.dot(b1_r[...], w2_r[...],
              preferred_element_type=jnp.float32) + b2_r[...]

  def bb_body(b, _):
    su = jnp.dot(us_r[b], ue_r[b], preferred_element_type=jnp.float32)
    si = jnp.dot(is_r[b], ie_r[b], preferred_element_type=jnp.float32)
    inter = su * si
    orow = lax.dot_general(wv, inter, (((0,), (1,)), ((), ())),
                           preferred_element_type=jnp.float32)
    acc_r[pl.ds(b, 1), :] = jax.nn.sigmoid(orow + c)
    return ()

  lax.fori_loop(0, bb, bb_body, (), unroll=8)
  o_r[...] = jnp.mean(acc_r[...], axis=1)


def _tc_compute(B, K, D, H, us, ue, is_, ie, W1, b1, W2, b2):
  bb = 128
  grid = (B // bb,)
  return pl.pallas_call(
      _tc_body,
      grid=grid,
      in_specs=[
          pl.BlockSpec((bb, K, K), lambda i: (i, 0, 0)),
          pl.BlockSpec((bb, K, D), lambda i: (i, 0, 0)),
          pl.BlockSpec((bb, K, K), lambda i: (i, 0, 0)),
          pl.BlockSpec((bb, K, D), lambda i: (i, 0, 0)),
          pl.BlockSpec((D, H), lambda i: (0, 0)),
          pl.BlockSpec((1, H), lambda i: (0, 0)),
          pl.BlockSpec((H, 1), lambda i: (0, 0)),
          pl.BlockSpec((1, 1), lambda i: (0, 0)),
      ],
      out_specs=pl.BlockSpec((bb,), lambda i: (i,)),
      out_shape=jax.ShapeDtypeStruct((B,), jnp.float32),
      scratch_shapes=[pltpu.VMEM((bb, K), jnp.float32)],
  )(us, ue, is_, ie, W1, b1, W2, b2)


def kernel(user_idxs, item_idxs, user_idx_tensor, user_scr_tensor,
           item_idx_tensor, item_scr_tensor, user_emb_table, item_emb_table,
           W1, b1, W2, b2):
  B = user_idxs.shape[0]
  K = user_idx_tensor.shape[1]
  D = user_emb_table.shape[1]
  H = W1.shape[1]
  un3, in3 = _sc_neighbors(B, K, user_idxs, item_idxs, user_idx_tensor,
                           item_idx_tensor)
  un = un3.reshape(-1)
  inn = in3.reshape(-1)
  ue = jnp.take(user_emb_table, un, axis=0)
  us = jnp.take(user_scr_tensor, un, axis=0)
  ie = jnp.take(item_emb_table, inn, axis=0)
  is_ = jnp.take(item_scr_tensor, inn, axis=0)
  return _tc_compute(B, K, D, H,
                     us.reshape(B, K, K), ue.reshape(B, K, D),
                     is_.reshape(B, K, K), ie.reshape(B, K, D),
                     W1, b1.reshape(1, H), W2, b2.reshape(1, 1))


# speedup vs baseline: 2.4261x; 2.4261x over previous
import functools

import jax
import jax.numpy as jnp
from jax import lax
from jax.experimental import pallas as pl
from jax.experimental.pallas import tpu as pltpu
from jax.experimental.pallas import tpu_sc as plsc

NC = 2
NS = 16
NW = NC * NS
CH = 128


def _sc_neighbors(B, K, user_idxs, item_idxs, uidx_t, iidx_t):
  bpw = B // NW
  mesh = plsc.VectorSubcoreMesh(core_axis_name="c", subcore_axis_name="s")

  @functools.partial(
      pl.kernel,
      mesh=mesh,
      compiler_params=pltpu.CompilerParams(use_tc_tiling_on_sc=False),
      out_type=[
          jax.ShapeDtypeStruct((B, K), jnp.int32),
          jax.ShapeDtypeStruct((B, K), jnp.int32),
      ],
      scratch_types=[
          pltpu.VMEM((bpw,), jnp.int32),
          pltpu.VMEM((bpw,), jnp.int32),
          pltpu.VMEM((bpw, K), jnp.int32),
          pltpu.VMEM((bpw, K), jnp.int32),
          pltpu.SemaphoreType.DMA,
      ],
  )
  def k(uids_h, iids_h, uidx_h, iidx_h, un_out, in_out,
        ub_v, ib_v, un_v, in_v, sem):
    wid = lax.axis_index("s") * NC + lax.axis_index("c")
    base = wid * bpw
    pltpu.sync_copy(uids_h.at[pl.ds(base, bpw)], ub_v)
    pltpu.sync_copy(iids_h.at[pl.ds(base, bpw)], ib_v)
    c1 = pltpu.async_copy(uidx_h.at[ub_v], un_v, sem)
    c2 = pltpu.async_copy(iidx_h.at[ib_v], in_v, sem)
    c1.wait()
    c2.wait()
    pltpu.sync_copy(un_v, un_out.at[pl.ds(base, bpw)])
    pltpu.sync_copy(in_v, in_out.at[pl.ds(base, bpw)])

  return k(user_idxs, item_idxs, uidx_t, iidx_t)


def _sc_rows(B, K, D, un_flat, in_flat, uemb_t, uscr_t, iemb_t, iscr_t):
  R = B * K
  rpw = R // NW
  n_ch = rpw // CH
  mesh = plsc.VectorSubcoreMesh(core_axis_name="c", subcore_axis_name="s")

  @functools.partial(
      pl.kernel,
      mesh=mesh,
      compiler_params=pltpu.CompilerParams(use_tc_tiling_on_sc=False),
      out_type=[
          jax.ShapeDtypeStruct((R, D), jnp.float32),
          jax.ShapeDtypeStruct((R, K), jnp.float32),
          jax.ShapeDtypeStruct((R, D), jnp.float32),
          jax.ShapeDtypeStruct((R, K), jnp.float32),
      ],
      scratch_types=[
          pltpu.VMEM((CH,), jnp.int32),
          pltpu.VMEM((CH,), jnp.int32),
          pltpu.VMEM((CH, D), jnp.float32),
          pltpu.VMEM((CH, K), jnp.float32),
          pltpu.VMEM((CH, D), jnp.float32),
          pltpu.VMEM((CH, K), jnp.float32),
          pltpu.SemaphoreType.DMA,
      ],
  )
  def k(un_h, in_h, uemb_h, uscr_h, iemb_h, iscr_h,
        ue_out, us_out, ie_out, is_out,
        ui_v, ii_v, ue_b, us_b, ie_b, is_b, sem):
    wid = lax.axis_index("s") * NC + lax.axis_index("c")
    base = wid * rpw

    for p in range(n_ch):
      off = base + p * CH
      pltpu.sync_copy(un_h.at[pl.ds(off, CH)], ui_v)
      pltpu.sync_copy(in_h.at[pl.ds(off, CH)], ii_v)
      c1 = pltpu.async_copy(uemb_h.at[ui_v], ue_b, sem)
      c2 = pltpu.async_copy(uscr_h.at[ui_v], us_b, sem)
      c3 = pltpu.async_copy(iemb_h.at[ii_v], ie_b, sem)
      c4 = pltpu.async_copy(iscr_h.at[ii_v], is_b, sem)
      c1.wait()
      c2.wait()
      c3.wait()
      c4.wait()
      pltpu.sync_copy(ue_b, ue_out.at[pl.ds(off, CH)])
      pltpu.sync_copy(us_b, us_out.at[pl.ds(off, CH)])
      pltpu.sync_copy(ie_b, ie_out.at[pl.ds(off, CH)])
      pltpu.sync_copy(is_b, is_out.at[pl.ds(off, CH)])

  return k(un_flat, in_flat, uemb_t, uscr_t, iemb_t, iscr_t)


def _tc_body(us_r, ue_r, is_r, ie_r, w1_r, b1_r, w2_r, b2_r, o_r, acc_r):
  bb = us_r.shape[0]
  wv = jnp.dot(w1_r[...], w2_r[...], preferred_element_type=jnp.float32)
  c = jnp.dot(b1_r[...], w2_r[...],
              preferred_element_type=jnp.float32) + b2_r[...]

  def bb_body(b, _):
    su = jnp.dot(us_r[b], ue_r[b], preferred_element_type=jnp.float32)
    si = jnp.dot(is_r[b], ie_r[b], preferred_element_type=jnp.float32)
    inter = su * si
    orow = lax.dot_general(wv, inter, (((0,), (1,)), ((), ())),
                           preferred_element_type=jnp.float32)
    acc_r[pl.ds(b, 1), :] = jax.nn.sigmoid(orow + c)
    return ()

  lax.fori_loop(0, bb, bb_body, (), unroll=8)
  o_r[...] = jnp.mean(acc_r[...], axis=1)


def _tc_compute(B, K, D, H, us, ue, is_, ie, W1, b1, W2, b2):
  bb = 128
  grid = (B // bb,)
  return pl.pallas_call(
      _tc_body,
      grid=grid,
      in_specs=[
          pl.BlockSpec((bb, K, K), lambda i: (i, 0, 0)),
          pl.BlockSpec((bb, K, D), lambda i: (i, 0, 0)),
          pl.BlockSpec((bb, K, K), lambda i: (i, 0, 0)),
          pl.BlockSpec((bb, K, D), lambda i: (i, 0, 0)),
          pl.BlockSpec((D, H), lambda i: (0, 0)),
          pl.BlockSpec((1, H), lambda i: (0, 0)),
          pl.BlockSpec((H, 1), lambda i: (0, 0)),
          pl.BlockSpec((1, 1), lambda i: (0, 0)),
      ],
      out_specs=pl.BlockSpec((bb,), lambda i: (i,)),
      out_shape=jax.ShapeDtypeStruct((B,), jnp.float32),
      scratch_shapes=[pltpu.VMEM((bb, K), jnp.float32)],
  )(us, ue, is_, ie, W1, b1, W2, b2)


def kernel(user_idxs, item_idxs, user_idx_tensor, user_scr_tensor,
           item_idx_tensor, item_scr_tensor, user_emb_table, item_emb_table,
           W1, b1, W2, b2):
  B = user_idxs.shape[0]
  K = user_idx_tensor.shape[1]
  D = user_emb_table.shape[1]
  H = W1.shape[1]
  un = jnp.take(user_idx_tensor, user_idxs, axis=0).reshape(-1)
  inn = jnp.take(item_idx_tensor, item_idxs, axis=0).reshape(-1)
  ue, us, ie, is_ = _sc_rows(B, K, D, un, inn,
                             user_emb_table, user_scr_tensor,
                             item_emb_table, item_scr_tensor)
  return _tc_compute(B, K, D, H,
                     us.reshape(B, K, K), ue.reshape(B, K, D),
                     is_.reshape(B, K, K), ie.reshape(B, K, D),
                     W1, b1.reshape(1, H), W2, b2.reshape(1, 1))


# speedup vs baseline: 2.5152x; 1.0367x over previous
import functools

import jax
import jax.numpy as jnp
from jax import lax
from jax.experimental import pallas as pl
from jax.experimental.pallas import tpu as pltpu
from jax.experimental.pallas import tpu_sc as plsc

NC = 2
NS = 16
NW = NC * NS
CH = 128


def _sc_neighbors(B, K, user_idxs, item_idxs, uidx_t, iidx_t):
  bpw = B // NW
  mesh = plsc.VectorSubcoreMesh(core_axis_name="c", subcore_axis_name="s")

  @functools.partial(
      pl.kernel,
      mesh=mesh,
      compiler_params=pltpu.CompilerParams(use_tc_tiling_on_sc=False),
      out_type=[
          jax.ShapeDtypeStruct((B, K), jnp.int32),
          jax.ShapeDtypeStruct((B, K), jnp.int32),
      ],
      scratch_types=[
          pltpu.VMEM((bpw,), jnp.int32),
          pltpu.VMEM((bpw,), jnp.int32),
          pltpu.VMEM((bpw, K), jnp.int32),
          pltpu.VMEM((bpw, K), jnp.int32),
          pltpu.SemaphoreType.DMA,
      ],
  )
  def k(uids_h, iids_h, uidx_h, iidx_h, un_out, in_out,
        ub_v, ib_v, un_v, in_v, sem):
    wid = lax.axis_index("s") * NC + lax.axis_index("c")
    base = wid * bpw
    pltpu.sync_copy(uids_h.at[pl.ds(base, bpw)], ub_v)
    pltpu.sync_copy(iids_h.at[pl.ds(base, bpw)], ib_v)
    c1 = pltpu.async_copy(uidx_h.at[ub_v], un_v, sem)
    c2 = pltpu.async_copy(iidx_h.at[ib_v], in_v, sem)
    c1.wait()
    c2.wait()
    pltpu.sync_copy(un_v, un_out.at[pl.ds(base, bpw)])
    pltpu.sync_copy(in_v, in_out.at[pl.ds(base, bpw)])

  return k(user_idxs, item_idxs, uidx_t, iidx_t)


def _sc_rows(B, K, D, un_flat, in_flat, uemb_t, uscr_t, iemb_t, iscr_t):
  R = B * K
  rpw = R // NW
  n_ch = rpw // CH
  mesh = plsc.VectorSubcoreMesh(core_axis_name="c", subcore_axis_name="s")

  @functools.partial(
      pl.kernel,
      mesh=mesh,
      compiler_params=pltpu.CompilerParams(use_tc_tiling_on_sc=False),
      out_type=[
          jax.ShapeDtypeStruct((R, D), jnp.float32),
          jax.ShapeDtypeStruct((R, K), jnp.float32),
          jax.ShapeDtypeStruct((R, D), jnp.float32),
          jax.ShapeDtypeStruct((R, K), jnp.float32),
      ],
      scratch_types=[
          pltpu.VMEM((CH,), jnp.int32),
          pltpu.VMEM((CH,), jnp.int32),
          pltpu.VMEM((CH, D), jnp.float32),
          pltpu.VMEM((CH, K), jnp.float32),
          pltpu.VMEM((CH, D), jnp.float32),
          pltpu.VMEM((CH, K), jnp.float32),
          pltpu.SemaphoreType.DMA,
      ],
  )
  def k(un_h, in_h, uemb_h, uscr_h, iemb_h, iscr_h,
        ue_out, us_out, ie_out, is_out,
        ui_v, ii_v, ue_b, us_b, ie_b, is_b, sem):
    wid = lax.axis_index("s") * NC + lax.axis_index("c")
    base = wid * rpw

    for p in range(n_ch):
      off = base + p * CH
      pltpu.sync_copy(un_h.at[pl.ds(off, CH)], ui_v)
      pltpu.sync_copy(in_h.at[pl.ds(off, CH)], ii_v)
      c1 = pltpu.async_copy(uemb_h.at[ui_v], ue_b, sem)
      c2 = pltpu.async_copy(uscr_h.at[ui_v], us_b, sem)
      c3 = pltpu.async_copy(iemb_h.at[ii_v], ie_b, sem)
      c4 = pltpu.async_copy(iscr_h.at[ii_v], is_b, sem)
      c1.wait()
      c2.wait()
      c3.wait()
      c4.wait()
      pltpu.sync_copy(ue_b, ue_out.at[pl.ds(off, CH)])
      pltpu.sync_copy(us_b, us_out.at[pl.ds(off, CH)])
      pltpu.sync_copy(ie_b, ie_out.at[pl.ds(off, CH)])
      pltpu.sync_copy(is_b, is_out.at[pl.ds(off, CH)])

  return k(un_flat, in_flat, uemb_t, uscr_t, iemb_t, iscr_t)


def _tc_body(us_r, ue_r, is_r, ie_r, w1_r, b1_r, w2_r, b2_r, o_r, acc_r):
  bb = us_r.shape[0]
  wv = jnp.dot(w1_r[...], w2_r[...], preferred_element_type=jnp.float32)
  c = jnp.dot(b1_r[...], w2_r[...],
              preferred_element_type=jnp.float32) + b2_r[...]

  def bb_body(b, _):
    su = jnp.dot(us_r[b], ue_r[b], preferred_element_type=jnp.float32)
    si = jnp.dot(is_r[b], ie_r[b], preferred_element_type=jnp.float32)
    inter = su * si
    orow = lax.dot_general(wv, inter, (((0,), (1,)), ((), ())),
                           preferred_element_type=jnp.float32)
    acc_r[pl.ds(b, 1), :] = jax.nn.sigmoid(orow + c)
    return ()

  lax.fori_loop(0, bb, bb_body, (), unroll=8)
  o_r[...] = jnp.mean(acc_r[...], axis=1)


def _tc_compute(B, K, D, H, us, ue, is_, ie, W1, b1, W2, b2):
  bb = 128
  grid = (B // bb,)
  return pl.pallas_call(
      _tc_body,
      grid=grid,
      in_specs=[
          pl.BlockSpec((bb, K, K), lambda i: (i, 0, 0)),
          pl.BlockSpec((bb, K, D), lambda i: (i, 0, 0)),
          pl.BlockSpec((bb, K, K), lambda i: (i, 0, 0)),
          pl.BlockSpec((bb, K, D), lambda i: (i, 0, 0)),
          pl.BlockSpec((D, H), lambda i: (0, 0)),
          pl.BlockSpec((1, H), lambda i: (0, 0)),
          pl.BlockSpec((H, 1), lambda i: (0, 0)),
          pl.BlockSpec((1, 1), lambda i: (0, 0)),
      ],
      out_specs=pl.BlockSpec((bb,), lambda i: (i,)),
      out_shape=jax.ShapeDtypeStruct((B,), jnp.float32),
      scratch_shapes=[pltpu.VMEM((bb, K), jnp.float32)],
  )(us, ue, is_, ie, W1, b1, W2, b2)


def kernel(user_idxs, item_idxs, user_idx_tensor, user_scr_tensor,
           item_idx_tensor, item_scr_tensor, user_emb_table, item_emb_table,
           W1, b1, W2, b2):
  B = user_idxs.shape[0]
  K = user_idx_tensor.shape[1]
  D = user_emb_table.shape[1]
  H = W1.shape[1]
  un = jnp.take(user_idx_tensor, user_idxs, axis=0).reshape(-1)
  inn = jnp.take(item_idx_tensor, item_idxs, axis=0).reshape(-1)
  n_split = 2
  Bc = B // n_split
  outs = []
  for s in range(n_split):
    lo = s * Bc * K
    ue, us, ie, is_ = _sc_rows(Bc, K, D,
                               lax.slice(un, (lo,), (lo + Bc * K,)),
                               lax.slice(inn, (lo,), (lo + Bc * K,)),
                               user_emb_table, user_scr_tensor,
                               item_emb_table, item_scr_tensor)
    outs.append(_tc_compute(Bc, K, D, H,
                            us.reshape(Bc, K, K), ue.reshape(Bc, K, D),
                            is_.reshape(Bc, K, K), ie.reshape(Bc, K, D),
                            W1, b1.reshape(1, H), W2, b2.reshape(1, 1)))
  return jnp.concatenate(outs)


# speedup vs baseline: 2.5341x; 1.0075x over previous
import functools

import jax
import jax.numpy as jnp
from jax import lax
from jax.experimental import pallas as pl
from jax.experimental.pallas import tpu as pltpu
from jax.experimental.pallas import tpu_sc as plsc

NC = 2
NS = 16
NW = NC * NS
CH = 128


def _sc_neighbors(B, K, user_idxs, item_idxs, uidx_t, iidx_t):
  bpw = B // NW
  mesh = plsc.VectorSubcoreMesh(core_axis_name="c", subcore_axis_name="s")

  @functools.partial(
      pl.kernel,
      mesh=mesh,
      compiler_params=pltpu.CompilerParams(use_tc_tiling_on_sc=False),
      out_type=[
          jax.ShapeDtypeStruct((B, K), jnp.int32),
          jax.ShapeDtypeStruct((B, K), jnp.int32),
      ],
      scratch_types=[
          pltpu.VMEM((bpw,), jnp.int32),
          pltpu.VMEM((bpw,), jnp.int32),
          pltpu.VMEM((bpw, K), jnp.int32),
          pltpu.VMEM((bpw, K), jnp.int32),
          pltpu.SemaphoreType.DMA,
      ],
  )
  def k(uids_h, iids_h, uidx_h, iidx_h, un_out, in_out,
        ub_v, ib_v, un_v, in_v, sem):
    wid = lax.axis_index("s") * NC + lax.axis_index("c")
    base = wid * bpw
    pltpu.sync_copy(uids_h.at[pl.ds(base, bpw)], ub_v)
    pltpu.sync_copy(iids_h.at[pl.ds(base, bpw)], ib_v)
    c1 = pltpu.async_copy(uidx_h.at[ub_v], un_v, sem)
    c2 = pltpu.async_copy(iidx_h.at[ib_v], in_v, sem)
    c1.wait()
    c2.wait()
    pltpu.sync_copy(un_v, un_out.at[pl.ds(base, bpw)])
    pltpu.sync_copy(in_v, in_out.at[pl.ds(base, bpw)])

  return k(user_idxs, item_idxs, uidx_t, iidx_t)


def _sc_rows(B, K, D, un_flat, in_flat, uemb_t, uscr_t, iemb_t, iscr_t):
  R = B * K
  rpw = R // NW
  n_ch = rpw // CH
  mesh = plsc.VectorSubcoreMesh(core_axis_name="c", subcore_axis_name="s")

  @functools.partial(
      pl.kernel,
      mesh=mesh,
      compiler_params=pltpu.CompilerParams(use_tc_tiling_on_sc=False),
      out_type=[
          jax.ShapeDtypeStruct((R, D), jnp.float32),
          jax.ShapeDtypeStruct((R, K), jnp.float32),
          jax.ShapeDtypeStruct((R, D), jnp.float32),
          jax.ShapeDtypeStruct((R, K), jnp.float32),
      ],
      scratch_types=[
          pltpu.VMEM((2, CH), jnp.int32),
          pltpu.VMEM((2, CH), jnp.int32),
          pltpu.VMEM((2, CH, D), jnp.float32),
          pltpu.VMEM((2, CH, K), jnp.float32),
          pltpu.VMEM((2, CH, D), jnp.float32),
          pltpu.VMEM((2, CH, K), jnp.float32),
          pltpu.SemaphoreType.DMA,
          pltpu.SemaphoreType.DMA,
      ],
  )
  def k(un_h, in_h, uemb_h, uscr_h, iemb_h, iscr_h,
        ue_out, us_out, ie_out, is_out,
        ui_v, ii_v, ue_b, us_b, ie_b, is_b, gsem, wsem):
    wid = lax.axis_index("s") * NC + lax.axis_index("c")
    base = wid * rpw

    def fire(p):
      s = p % 2
      off = base + p * CH
      pltpu.sync_copy(un_h.at[pl.ds(off, CH)], ui_v.at[s])
      pltpu.sync_copy(in_h.at[pl.ds(off, CH)], ii_v.at[s])
      return [
          pltpu.async_copy(uemb_h.at[ui_v.at[s]], ue_b.at[s], gsem),
          pltpu.async_copy(uscr_h.at[ui_v.at[s]], us_b.at[s], gsem),
          pltpu.async_copy(iemb_h.at[ii_v.at[s]], ie_b.at[s], gsem),
          pltpu.async_copy(iscr_h.at[ii_v.at[s]], is_b.at[s], gsem),
      ]

    def write(p):
      s = p % 2
      off = base + p * CH
      return [
          pltpu.async_copy(ue_b.at[s], ue_out.at[pl.ds(off, CH)], wsem),
          pltpu.async_copy(us_b.at[s], us_out.at[pl.ds(off, CH)], wsem),
          pltpu.async_copy(ie_b.at[s], ie_out.at[pl.ds(off, CH)], wsem),
          pltpu.async_copy(is_b.at[s], is_out.at[pl.ds(off, CH)], wsem),
      ]

    gath = {0: fire(0)}
    wrts = {}
    for p in range(n_ch):
      if p + 1 < n_ch:
        if p >= 1:
          for c in wrts.pop(p - 1):
            c.wait()
        gath[p + 1] = fire(p + 1)
      for c in gath.pop(p):
        c.wait()
      wrts[p] = write(p)
    for p in sorted(wrts):
      for c in wrts.pop(p):
        c.wait()

  return k(un_flat, in_flat, uemb_t, uscr_t, iemb_t, iscr_t)


def _tc_body(us_r, ue_r, is_r, ie_r, w1_r, b1_r, w2_r, b2_r, o_r, acc_r):
  bb = us_r.shape[0]
  wv = jnp.dot(w1_r[...], w2_r[...], preferred_element_type=jnp.float32)
  c = jnp.dot(b1_r[...], w2_r[...],
              preferred_element_type=jnp.float32) + b2_r[...]

  def bb_body(b, _):
    su = jnp.dot(us_r[b], ue_r[b], preferred_element_type=jnp.float32)
    si = jnp.dot(is_r[b], ie_r[b], preferred_element_type=jnp.float32)
    inter = su * si
    orow = lax.dot_general(wv, inter, (((0,), (1,)), ((), ())),
                           preferred_element_type=jnp.float32)
    acc_r[pl.ds(b, 1), :] = jax.nn.sigmoid(orow + c)
    return ()

  lax.fori_loop(0, bb, bb_body, (), unroll=8)
  o_r[...] = jnp.mean(acc_r[...], axis=1)


def _tc_compute(B, K, D, H, us, ue, is_, ie, W1, b1, W2, b2):
  bb = 128
  grid = (B // bb,)
  return pl.pallas_call(
      _tc_body,
      grid=grid,
      in_specs=[
          pl.BlockSpec((bb, K, K), lambda i: (i, 0, 0)),
          pl.BlockSpec((bb, K, D), lambda i: (i, 0, 0)),
          pl.BlockSpec((bb, K, K), lambda i: (i, 0, 0)),
          pl.BlockSpec((bb, K, D), lambda i: (i, 0, 0)),
          pl.BlockSpec((D, H), lambda i: (0, 0)),
          pl.BlockSpec((1, H), lambda i: (0, 0)),
          pl.BlockSpec((H, 1), lambda i: (0, 0)),
          pl.BlockSpec((1, 1), lambda i: (0, 0)),
      ],
      out_specs=pl.BlockSpec((bb,), lambda i: (i,)),
      out_shape=jax.ShapeDtypeStruct((B,), jnp.float32),
      scratch_shapes=[pltpu.VMEM((bb, K), jnp.float32)],
  )(us, ue, is_, ie, W1, b1, W2, b2)


def kernel(user_idxs, item_idxs, user_idx_tensor, user_scr_tensor,
           item_idx_tensor, item_scr_tensor, user_emb_table, item_emb_table,
           W1, b1, W2, b2):
  B = user_idxs.shape[0]
  K = user_idx_tensor.shape[1]
  D = user_emb_table.shape[1]
  H = W1.shape[1]
  un = jnp.take(user_idx_tensor, user_idxs, axis=0).reshape(-1)
  inn = jnp.take(item_idx_tensor, item_idxs, axis=0).reshape(-1)
  n_split = 2
  Bc = B // n_split
  outs = []
  for s in range(n_split):
    lo = s * Bc * K
    ue, us, ie, is_ = _sc_rows(Bc, K, D,
                               lax.slice(un, (lo,), (lo + Bc * K,)),
                               lax.slice(inn, (lo,), (lo + Bc * K,)),
                               user_emb_table, user_scr_tensor,
                               item_emb_table, item_scr_tensor)
    outs.append(_tc_compute(Bc, K, D, H,
                            us.reshape(Bc, K, K), ue.reshape(Bc, K, D),
                            is_.reshape(Bc, K, K), ie.reshape(Bc, K, D),
                            W1, b1.reshape(1, H), W2, b2.reshape(1, 1)))
  return jnp.concatenate(outs)
